# baseline (speedup 1.0000x reference)
import functools
import math

import jax
import jax.numpy as jnp
from jax import lax
from jax.experimental import pallas as pl
from jax.experimental.pallas import tpu as pltpu
from jax.experimental.pallas import tpu_sc as plsc

N_LEVELS = 16
F = 2
LOG2_T = 19
T = 1 << LOG2_T
MASK = T - 1
BASE_RES = 16
FINEST_RES = 2048
PER_LEVEL_SCALE = math.exp((math.log(FINEST_RES) - math.log(BASE_RES)) / (N_LEVELS - 1))
P1 = ((2654435761 + (1 << 31)) % (1 << 32)) - (1 << 31)
P2 = ((805459861 + (1 << 31)) % (1 << 32)) - (1 << 31)

NC = 2
NS = 16
LANES = 16
NW = NC * NS

PB = 128
CHUNK = 128
NSTR = 2 * 8

_SCALES = []
_RES = []
_DENSE = []
for _l in range(N_LEVELS):
    _s = BASE_RES * (PER_LEVEL_SCALE ** _l) - 1.0
    _r = int(math.ceil(_s)) + 1
    _SCALES.append(_s)
    _RES.append(_r)
    _DENSE.append(_r ** 3 <= T)


def _corner_terms(level, xi, yi, zi):
    if _DENSE[level]:
        r = _RES[level]
        mx, my, mz = 1, r, r * r
    else:
        mx, my, mz = 1, P1, P2
    xs = (xi, xi + jnp.int32(mx))
    ys = (yi * jnp.int32(my), yi * jnp.int32(my) + jnp.int32(my))
    zs = (zi * jnp.int32(mz), zi * jnp.int32(mz) + jnp.int32(mz))
    return xs, ys, zs


def _body(px_hbm, py_hbm, pz_hbm, lob_hbm, denb_hbm, tbl_hbm, out_hbm,
          lov, denv, pbuf, fracb, idxb, rowsf, outt, sem_g):
    n = px_hbm.shape[0]
    per_w = n // NW
    nblk = per_w // PB
    wid = lax.axis_index("s") * NC + lax.axis_index("c")
    base0 = wid * per_w
    pltpu.sync_copy(lob_hbm, lov)
    pltpu.sync_copy(denb_hbm, denv)

    @pl.loop(0, nblk)
    def _blk(blk):
        base = base0 + blk * PB
        for d, ref in enumerate((px_hbm, py_hbm, pz_hbm)):
            pltpu.sync_copy(ref.at[pl.ds(base, PB)], pbuf.at[d])

        @pl.loop(0, PB // LANES)
        def _norm(g):
            s = pl.ds(g * LANES, LANES)
            for d in range(3):
                x = (pbuf[d, s] - lov[d, :]) / denv[d, :]
                pbuf[d, s] = jnp.minimum(
                    jnp.maximum(x, jnp.float32(0.0)), jnp.float32(1.0))

        for level in range(N_LEVELS):
            scale = jnp.float32(_SCALES[level])
            lt2 = jnp.int32(2 * level * T)

            @pl.loop(0, PB // LANES)
            def _ixg(g, level=level, scale=scale, lt2=lt2):
                s = pl.ds(g * LANES, LANES)
                ints = []
                for d in range(3):
                    pos = pbuf[d, s] * scale + jnp.float32(0.5)
                    ii = pos.astype(jnp.int32)
                    fracb[d, s] = pos - ii.astype(jnp.float32)
                    ints.append(ii)
                xs, ys, zs = _corner_terms(level, *ints)
                for c in range(8):
                    cx, cy, cz = c & 1, (c >> 1) & 1, (c >> 2) & 1
                    if _DENSE[level]:
                        idx = xs[cx] + ys[cy] + zs[cz]
                    else:
                        idx = xs[cx] ^ ys[cy] ^ zs[cz]
                    f0 = ((idx & jnp.int32(MASK)) << 1) + lt2
                    idxb[2 * c, s] = f0
                    idxb[2 * c + 1, s] = f0 + jnp.int32(1)

            cps = []
            for r in range(NSTR):
                for k in range(PB // CHUNK):
                    cps.append(pltpu.async_copy(
                        tbl_hbm.at[idxb.at[r, pl.ds(k * CHUNK, CHUNK)]],
                        rowsf.at[r, pl.ds(k * CHUNK, CHUNK)], sem_g))
            for cp in cps:
                cp.wait()

            @pl.loop(0, PB // LANES)
            def _acc(g, level=level):
                s = pl.ds(g * LANES, LANES)
                fx = fracb[0, s]
                fy = fracb[1, s]
                fz = fracb[2, s]
                one = jnp.float32(1.0)
                wx = (one - fx, fx)
                wy = (one - fy, fy)
                wz = (one - fz, fz)
                acc0 = acc1 = None
                for c in range(8):
                    cx, cy, cz = c & 1, (c >> 1) & 1, (c >> 2) & 1
                    w = wx[cx] * wy[cy] * wz[cz]
                    g0 = rowsf[2 * c, s]
                    g1 = rowsf[2 * c + 1, s]
                    acc0 = w * g0 if acc0 is None else acc0 + w * g0
                    acc1 = w * g1 if acc1 is None else acc1 + w * g1
                outt[2 * level, s] = acc0
                outt[2 * level + 1, s] = acc1

        for r in range(N_LEVELS * F):
            pltpu.sync_copy(outt.at[r], out_hbm.at[pl.ds(r * n + base, PB)])


@functools.lru_cache(maxsize=None)
def _make_kernel(n):
    mesh = plsc.VectorSubcoreMesh(core_axis_name="c", subcore_axis_name="s",
                                  num_cores=NC, num_subcores=NS)
    return pl.kernel(
        _body,
        out_type=jax.ShapeDtypeStruct((N_LEVELS * F * n,), jnp.float32),
        mesh=mesh,
        scratch_types=[
            pltpu.VMEM((3, LANES), jnp.float32),
            pltpu.VMEM((3, LANES), jnp.float32),
            pltpu.VMEM((3, PB), jnp.float32),
            pltpu.VMEM((3, PB), jnp.float32),
            pltpu.VMEM((NSTR, PB), jnp.int32),
            pltpu.VMEM((NSTR, PB), jnp.float32),
            pltpu.VMEM((N_LEVELS * F, PB), jnp.float32),
            pltpu.SemaphoreType.DMA,
        ],
    )


@jax.jit
def kernel(p, bound, table):
    in_shape = p.shape
    p2 = p.reshape(-1, 3)
    n = p2.shape[0]
    px = p2[:, 0]
    py = p2[:, 1]
    pz = p2[:, 2]
    lo = bound[:, 0]
    den = bound[:, 1] - bound[:, 0]
    lob = jnp.broadcast_to(lo[:, None], (3, LANES))
    denb = jnp.broadcast_to(den[:, None], (3, LANES))
    tbl = table.reshape(N_LEVELS * T * F)
    out = _make_kernel(n)(px, py, pz, lob, denb, tbl)
    out = out.reshape(N_LEVELS * F, n).T
    return out.reshape(*in_shape[:-1], N_LEVELS * F)


# speedup vs baseline: 1.1178x; 1.1178x over previous
import functools
import math

import jax
import jax.numpy as jnp
from jax import lax
from jax.experimental import pallas as pl
from jax.experimental.pallas import tpu as pltpu
from jax.experimental.pallas import tpu_sc as plsc

N_LEVELS = 16
F = 2
LOG2_T = 19
T = 1 << LOG2_T
MASK = T - 1
BASE_RES = 16
FINEST_RES = 2048
PER_LEVEL_SCALE = math.exp((math.log(FINEST_RES) - math.log(BASE_RES)) / (N_LEVELS - 1))
P1 = ((2654435761 + (1 << 31)) % (1 << 32)) - (1 << 31)
P2 = ((805459861 + (1 << 31)) % (1 << 32)) - (1 << 31)

NC = 2
NS = 16
LANES = 16
NW = NC * NS

PB = 1024
NSTR = 2 * 8

_SCALES = []
_RES = []
_DENSE = []
for _l in range(N_LEVELS):
    _s = BASE_RES * (PER_LEVEL_SCALE ** _l) - 1.0
    _r = int(math.ceil(_s)) + 1
    _SCALES.append(_s)
    _RES.append(_r)
    _DENSE.append(_r ** 3 <= T)


def _corner_terms(level, xi, yi, zi):
    if _DENSE[level]:
        mx, my, mz = 1, _RES[level], _RES[level] ** 2
    else:
        mx, my, mz = 1, P1, P2
    xs = (xi, xi + jnp.int32(mx))
    ys = (yi * jnp.int32(my), yi * jnp.int32(my) + jnp.int32(my))
    zs = (zi * jnp.int32(mz), zi * jnp.int32(mz) + jnp.int32(mz))
    return xs, ys, zs


def _body(px_hbm, py_hbm, pz_hbm, lob_hbm, denb_hbm, tbl_hbm, out_hbm,
          lov, denv, pbuf, fracb, idxb0, idxb1, rows0, rows1, outt,
          sem0, sem1):
    n = px_hbm.shape[0]
    per_w = n // NW
    nblk = per_w // PB
    wid = lax.axis_index("s") * NC + lax.axis_index("c")
    base0 = wid * per_w
    pltpu.sync_copy(lob_hbm, lov)
    pltpu.sync_copy(denb_hbm, denv)
    sems = (sem0, sem1)
    idxbs = (idxb0, idxb1)
    rowss = (rows0, rows1)

    def idx_phase(level, buf):
        scale = jnp.float32(_SCALES[level])
        lt2 = jnp.int32(2 * level * T)

        @pl.loop(0, PB // LANES)
        def _ixg(g):
            s = pl.ds(g * LANES, LANES)
            ints = []
            for d in range(3):
                pos = pbuf[pl.ds(d * PB + g * LANES, LANES)] * scale + jnp.float32(0.5)
                ii = pos.astype(jnp.int32)
                fracb[3 * buf + d, s] = pos - ii.astype(jnp.float32)
                ints.append(ii)
            xs, ys, zs = _corner_terms(level, *ints)
            for c in range(8):
                cx, cy, cz = c & 1, (c >> 1) & 1, (c >> 2) & 1
                if _DENSE[level]:
                    idx = xs[cx] + ys[cy] + zs[cz]
                else:
                    idx = xs[cx] ^ ys[cy] ^ zs[cz]
                f0 = ((idx & jnp.int32(MASK)) << 1) + lt2
                idxbs[buf][pl.ds((2 * c) * PB + g * LANES, LANES)] = f0
                idxbs[buf][pl.ds((2 * c + 1) * PB + g * LANES, LANES)] = (
                    f0 + jnp.int32(1))

        return pltpu.async_copy(tbl_hbm.at[idxbs[buf]], rowss[buf], sems[buf])

    def acc_phase(level, buf):
        @pl.loop(0, PB // LANES)
        def _acc(g):
            s = pl.ds(g * LANES, LANES)
            fx = fracb[3 * buf + 0, s]
            fy = fracb[3 * buf + 1, s]
            fz = fracb[3 * buf + 2, s]
            one = jnp.float32(1.0)
            wx = (one - fx, fx)
            wy = (one - fy, fy)
            wz = (one - fz, fz)
            acc0 = acc1 = None
            for c in range(8):
                cx, cy, cz = c & 1, (c >> 1) & 1, (c >> 2) & 1
                w = wx[cx] * wy[cy] * wz[cz]
                g0 = rowss[buf][pl.ds((2 * c) * PB + g * LANES, LANES)]
                g1 = rowss[buf][pl.ds((2 * c + 1) * PB + g * LANES, LANES)]
                acc0 = w * g0 if acc0 is None else acc0 + w * g0
                acc1 = w * g1 if acc1 is None else acc1 + w * g1
            outt[pl.ds((2 * level) * PB + g * LANES, LANES)] = acc0
            outt[pl.ds((2 * level + 1) * PB + g * LANES, LANES)] = acc1

    @pl.loop(0, nblk)
    def _blk(blk):
        base = base0 + blk * PB
        for d, ref in enumerate((px_hbm, py_hbm, pz_hbm)):
            pltpu.sync_copy(ref.at[pl.ds(base, PB)], pbuf.at[pl.ds(d * PB, PB)])

        @pl.loop(0, PB // LANES)
        def _norm(g):
            s = pl.ds(g * LANES, LANES)
            for d in range(3):
                sd = pl.ds(d * PB + g * LANES, LANES)
                x = (pbuf[sd] - lov[d, :]) / denv[d, :]
                pbuf[sd] = jnp.minimum(
                    jnp.maximum(x, jnp.float32(0.0)), jnp.float32(1.0))

        cps = [None, None]
        cps[0] = idx_phase(0, 0)
        for level in range(1, N_LEVELS):
            buf = level % 2
            cps[buf] = idx_phase(level, buf)
            cps[1 - buf].wait()
            acc_phase(level - 1, 1 - buf)
        cps[1].wait()
        acc_phase(N_LEVELS - 1, 1)

        for r in range(N_LEVELS * F):
            pltpu.sync_copy(outt.at[pl.ds(r * PB, PB)],
                            out_hbm.at[pl.ds(r * n + base, PB)])


@functools.lru_cache(maxsize=None)
def _make_kernel(n):
    mesh = plsc.VectorSubcoreMesh(core_axis_name="c", subcore_axis_name="s",
                                  num_cores=NC, num_subcores=NS)
    return pl.kernel(
        _body,
        out_type=jax.ShapeDtypeStruct((N_LEVELS * F * n,), jnp.float32),
        mesh=mesh,
        scratch_types=[
            pltpu.VMEM((3, LANES), jnp.float32),
            pltpu.VMEM((3, LANES), jnp.float32),
            pltpu.VMEM((3 * PB,), jnp.float32),
            pltpu.VMEM((6, PB), jnp.float32),
            pltpu.VMEM((NSTR * PB,), jnp.int32),
            pltpu.VMEM((NSTR * PB,), jnp.int32),
            pltpu.VMEM((NSTR * PB,), jnp.float32),
            pltpu.VMEM((NSTR * PB,), jnp.float32),
            pltpu.VMEM((N_LEVELS * F * PB,), jnp.float32),
            pltpu.SemaphoreType.DMA,
            pltpu.SemaphoreType.DMA,
        ],
    )


@jax.jit
def kernel(p, bound, table):
    in_shape = p.shape
    p2 = p.reshape(-1, 3)
    n = p2.shape[0]
    px = p2[:, 0]
    py = p2[:, 1]
    pz = p2[:, 2]
    lo = bound[:, 0]
    den = bound[:, 1] - bound[:, 0]
    lob = jnp.broadcast_to(lo[:, None], (3, LANES))
    denb = jnp.broadcast_to(den[:, None], (3, LANES))
    tbl = table.reshape(N_LEVELS * T * F)
    out = _make_kernel(n)(px, py, pz, lob, denb, tbl)
    out = out.reshape(N_LEVELS * F, n).T
    return out.reshape(*in_shape[:-1], N_LEVELS * F)
